# baseline (speedup 1.0000x reference)
import functools

import jax
import jax.numpy as jnp
from jax import lax
from jax.experimental import pallas as pl
from jax.experimental.pallas import tpu as pltpu
from jax.experimental.pallas import tpu_sc as plsc

_B, _T, _C = 2, 2048, 768
_NUM_RWKV, _NUM_TRANS = 6, 2
_E = _NUM_RWKV + _NUM_TRANS
_N = _B * _T

_BLK1 = 512
_NB1 = _N // _BLK1

_F32 = jnp.float32
_BF16 = jnp.bfloat16


def _bdot(a, b):
    return lax.dot_general(a, b, (((1,), (0,)), ((), ())),
                           preferred_element_type=_F32)


def _ddot(a, b):
    return lax.dot_general(a, b, (((1,), (0,)), ((), ())),
                           precision=lax.Precision.DEFAULT,
                           preferred_element_type=_F32)


def _ln_rows(x, g, b):
    mu = jnp.mean(x, axis=1, keepdims=True)
    d = x - mu
    var = jnp.mean(d * d, axis=1, keepdims=True)
    return d / jnp.sqrt(var + 1e-5) * g + b


def _k1_body(x_ref, vf_ref, wr_ref, wk_ref, wv_ref, wo_ref, ws_ref,
             cw_ref, wa_ref, wd_ref, wb1_ref, wb2_ref,
             lnp_ref, cap_ref,
             x1_ref, hs_ref, ss_ref, win_ref, cost_ref, sse_ref):
    i = pl.program_id(0)
    xf = x_ref[...]
    xn = _ln_rows(xf, lnp_ref[0:1, :], lnp_ref[1:2, :])
    r = _ddot(xn, wr_ref[...])
    k = _ddot(xn, wk_ref[...])
    v0 = _ddot(xn, wv_ref[...])
    v = v0 + (vf_ref[...] - v0) * jax.nn.sigmoid(k)
    att = _ddot(jax.nn.sigmoid(r) * v, wo_ref[...])
    state = jnp.tanh(_ddot(xn, ws_ref[...]))
    x1 = xf + att
    x1_ref[...] = x1
    h = _ln_rows(x1, lnp_ref[2:3, :], lnp_ref[3:4, :])
    conf = jax.nn.sigmoid(_ddot(h, cw_ref[...]))
    aff = _ddot(h, wa_ref[...])
    dcol = _ddot(h, wd_ref[...])[:, 0:1]
    diff = jax.nn.softplus(dcol)
    eff = conf * diff + 0.1 * aff + jnp.log(cap_ref[...] + 1e-6)
    costs = jnp.max(eff, axis=1)
    ids = lax.broadcasted_iota(jnp.int32, (_BLK1, _E), 1)
    win = jnp.min(jnp.where(eff == costs[:, None], ids, _E), axis=1)
    win_ref[...] = win[:, None]
    cost_ref[...] = costs[:, None]
    wc = jnp.sum(jnp.where(ids == win[:, None], conf, 0.0), axis=1)
    scale = (wc / (wc + 1e-6))[:, None]
    hs_ref[...] = h * scale
    ss_ref[...] = state * scale
    t1 = jnp.tanh(_ddot(state, wb1_ref[...]))
    recon = _ddot(t1, wb2_ref[...])
    dsse = jnp.sum((recon - h) ** 2)

    @pl.when(i == 0)
    def _():
        sse_ref[0, 0] = dsse

    @pl.when(i != 0)
    def _():
        sse_ref[0, 0] += dsse


def _k1(x2d, vf2d, Wr, Wk, Wv, Wo, Ws, cwT, Wa, Wd8, Wb1, Wb2, lnp, cap):
    n = x2d.shape[0]
    row_spec = pl.BlockSpec((_BLK1, _C), lambda i: (i, 0))
    full = lambda a: pl.BlockSpec(a.shape, lambda i: tuple(0 for _ in a.shape))
    out_shapes = (
        jax.ShapeDtypeStruct((n, _C), _F32),
        jax.ShapeDtypeStruct((n, _C), _F32),
        jax.ShapeDtypeStruct((n, _C), _F32),
        jax.ShapeDtypeStruct((n, 1), jnp.int32),
        jax.ShapeDtypeStruct((n, 1), _F32),
        jax.ShapeDtypeStruct((1, 1), _F32),
    )
    out_specs = (
        row_spec,
        row_spec,
        row_spec,
        pl.BlockSpec((_BLK1, 1), lambda i: (i, 0)),
        pl.BlockSpec((_BLK1, 1), lambda i: (i, 0)),
        pl.BlockSpec((1, 1), lambda i: (0, 0), memory_space=pltpu.SMEM),
    )
    ws = [Wr, Wk, Wv, Wo, Ws, cwT, Wa, Wd8, Wb1, Wb2, lnp, cap]
    return pl.pallas_call(
        _k1_body,
        grid=(n // _BLK1,),
        in_specs=[row_spec, row_spec] + [full(w) for w in ws],
        out_specs=out_specs,
        out_shape=out_shapes,
        compiler_params=pltpu.CompilerParams(
            dimension_semantics=("arbitrary",)),
    )(x2d, vf2d, *ws)


_C2 = 128
_M4 = 256


def _make_k2(n):
    nr = n // _C2
    nb = n // _M4
    g4p = nb + _E

    def body(w_ref, pos_ref, offs_ref, bm_ref, exc_ref, wmx_ref,
             first_ref, exr_ref):
        w = w_ref[...]
        iu = lax.broadcasted_iota(jnp.int32, (_C2, _C2), 0)
        ju = lax.broadcasted_iota(jnp.int32, (_C2, _C2), 1)
        U = (iu < ju).astype(_BF16)
        il = lax.broadcasted_iota(jnp.int32, (nr, nr), 0)
        jl = lax.broadcasted_iota(jnp.int32, (nr, nr), 1)
        L = (jl < il).astype(_BF16)
        pos = jnp.zeros((nr, _C2), _F32)
        off = jnp.float32(0.0)
        for e in range(_E):
            m = (w == e).astype(_F32)
            pref = _bdot(m.astype(_BF16), U)
            tot = jnp.sum(m, axis=1, keepdims=True)
            rowpref = _bdot(L, tot.astype(_BF16))
            offs_ref[e] = off.astype(jnp.int32)
            pos = pos + m * (off + rowpref + pref)
            off = off + jnp.sum(m)
        for e in range(_E, 16):
            offs_ref[e] = jnp.int32(n)
        pos_ref[...] = pos.astype(jnp.int32)

        def sched(t, carry):
            idx, prevb = carry
            b = t // _E
            e = t - b * _E
            lo = offs_ref[e]
            hi = offs_ref[e + 1]
            inc = jnp.logical_and(lo < (b + 1) * _M4, hi > b * _M4)

            @pl.when(inc)
            def _():
                bm_ref[idx] = b
                exr_ref[idx] = e
                exc_ref[idx] = e
                wmx_ref[idx] = jnp.clip(e - _NUM_RWKV, 0, 1)
                first_ref[idx] = jnp.where(b != prevb, 1, 0)

            return idx + inc.astype(jnp.int32), jnp.where(inc, b, prevb)

        idx, _ = lax.fori_loop(0, nb * _E, sched,
                               (jnp.int32(0), jnp.int32(-1)))

        def pad(g, c):
            bm_ref[g] = nb - 1
            exr_ref[g] = _E
            exc_ref[g] = _E - 1
            wmx_ref[g] = 0
            first_ref[g] = 0
            return c

        lax.fori_loop(idx, g4p, pad, jnp.int32(0))

    smem1d = lambda m: pl.BlockSpec((m,), lambda i: (0,),
                                    memory_space=pltpu.SMEM)

    def call(win2d32):
        return pl.pallas_call(
            body,
            grid=(1,),
            in_specs=[pl.BlockSpec((nr, _C2), lambda i: (0, 0))],
            out_specs=(
                pl.BlockSpec((nr, _C2), lambda i: (0, 0)),
                smem1d(16), smem1d(g4p), smem1d(g4p), smem1d(g4p),
                smem1d(g4p), smem1d(g4p),
            ),
            out_shape=(
                jax.ShapeDtypeStruct((nr, _C2), jnp.int32),
                jax.ShapeDtypeStruct((16,), jnp.int32),
                jax.ShapeDtypeStruct((g4p,), jnp.int32),
                jax.ShapeDtypeStruct((g4p,), jnp.int32),
                jax.ShapeDtypeStruct((g4p,), jnp.int32),
                jax.ShapeDtypeStruct((g4p,), jnp.int32),
                jax.ShapeDtypeStruct((g4p,), jnp.int32),
            ),
        )(win2d32)

    return call


def _k4_body(bm_r, exc_r, wmx_r, first_r, exraw_r, offs_r,
             hs_r, ss_r, x1_r, w1_r, w2_r, wm_r, out_r, acc_r):
    g = pl.program_id(0)
    e = exraw_r[g]
    lo = offs_r[e]
    hi = offs_r[e + 1]
    istrans = e >= _NUM_RWKV

    @pl.when(istrans)
    def _():
        acc_r[...] = hs_r[...] + _ddot(ss_r[...], wm_r[0])

    @pl.when(jnp.logical_not(istrans))
    def _():
        acc_r[...] = hs_r[...]

    u = jax.nn.relu(_ddot(acc_r[...], w1_r[0]))
    eo = _ddot(u, w2_r[0])
    rows = bm_r[g] * _M4 + lax.broadcasted_iota(jnp.int32, (_M4, 1), 0)
    inb = jnp.logical_and(rows >= lo, rows < hi)
    contrib = jnp.where(inb, eo, 0.0)
    isfirst = first_r[g] == 1

    @pl.when(isfirst)
    def _():
        out_r[...] = x1_r[...] + contrib

    @pl.when(jnp.logical_not(isfirst))
    def _():
        out_r[...] += contrib


def _make_k4(n):
    nb = n // _M4
    g4 = nb + _E - 1

    def call(bm, exc, wmx, first, exraw, offs, hs_s, ss_s, x1_s,
             W1b, W2b, Wmb):
        row_spec = pl.BlockSpec((_M4, _C), lambda g, *s: (s[0][g], 0))
        grid_spec = pltpu.PrefetchScalarGridSpec(
            num_scalar_prefetch=6,
            grid=(g4,),
            in_specs=[
                row_spec, row_spec, row_spec,
                pl.BlockSpec((1, _C, _C), lambda g, *s: (s[1][g], 0, 0)),
                pl.BlockSpec((1, _C, _C), lambda g, *s: (s[1][g], 0, 0)),
                pl.BlockSpec((1, _C, _C), lambda g, *s: (s[2][g], 0, 0)),
            ],
            out_specs=pl.BlockSpec((_M4, _C), lambda g, *s: (s[0][g], 0)),
            scratch_shapes=[pltpu.VMEM((_M4, _C), _F32)],
        )
        return pl.pallas_call(
            _k4_body,
            grid_spec=grid_spec,
            out_shape=jax.ShapeDtypeStruct((n, _C), _F32),
            compiler_params=pltpu.CompilerParams(
                dimension_semantics=("arbitrary",)),
        )(bm, exc, wmx, first, exraw, offs, hs_s, ss_s, x1_s, W1b, W2b, Wmb)

    return call


_NC, _NS = 2, 16
_NW = _NC * _NS


def _make_k3(n):
    ch = n // _NW
    qc = ch // 4

    def body(pos_r, hs_r, ss_r, x1_r, hs_o, ss_o, x1_o,
             idx_v, b0, b1, f0, f1, sib0, sib1, sob0, sob1,
             sif0, sif1, sof0, sof1):
        wid = lax.axis_index("s") * _NC + lax.axis_index("c")
        base = wid * ch
        pltpu.sync_copy(pos_r.at[wid], idx_v)
        bbufs = (b0, b1)
        fbufs = (f0, f1)

        def inb(t):
            a = (hs_r, ss_r)[t // 4]
            return pltpu.make_async_copy(
                a.at[pl.ds(base + (t % 4) * qc, qc)],
                bbufs[t % 2], (sib0, sib1)[t % 2])

        def outb(t):
            d = (hs_o, ss_o)[t // 4]
            return pltpu.make_async_copy(
                bbufs[t % 2], d.at[idx_v.at[t % 4]], (sob0, sob1)[t % 2])

        def inf(t):
            return pltpu.make_async_copy(
                x1_r.at[pl.ds(base + t * qc, qc)],
                fbufs[t % 2], (sif0, sif1)[t % 2])

        def outf(t):
            return pltpu.make_async_copy(
                fbufs[t % 2], x1_o.at[idx_v.at[t]], (sof0, sof1)[t % 2])

        inb(0).start()
        inf(0).start()
        for t in range(8):
            inb(t).wait()
            if t >= 1:
                outb(t - 1).wait()
            if t + 1 < 8:
                inb(t + 1).start()
            outb(t).start()
        for t in range(4):
            inf(t).wait()
            if t >= 1:
                outf(t - 1).wait()
            if t + 1 < 4:
                inf(t + 1).start()
            outf(t).start()
        outb(7).wait()
        outf(3).wait()

    def call(pos3d, hs, ss, x1):
        mesh = plsc.VectorSubcoreMesh(core_axis_name="c",
                                      subcore_axis_name="s")
        sdsf = jax.ShapeDtypeStruct((n, _C), _F32)
        run = functools.partial(
            pl.kernel, mesh=mesh,
            out_type=(sdsf, sdsf, sdsf),
            scratch_types=[
                pltpu.VMEM((4, qc), jnp.int32),
                pltpu.VMEM((qc, _C), _F32),
                pltpu.VMEM((qc, _C), _F32),
                pltpu.VMEM((qc, _C), _F32),
                pltpu.VMEM((qc, _C), _F32),
            ] + [pltpu.SemaphoreType.DMA] * 8)(body)
        return run(pos3d, hs, ss, x1)

    return call


def _make_k5(n):
    ch = n // _NW

    def body(pos_r, os_r, xo_r, idx_v, buf, sem):
        wid = lax.axis_index("s") * _NC + lax.axis_index("c")
        base = wid * ch
        pltpu.sync_copy(pos_r.at[wid], idx_v)
        pltpu.async_copy(os_r.at[idx_v], buf, sem).wait()
        pltpu.sync_copy(buf, xo_r.at[pl.ds(base, ch)])

    def call(pos2d, out_sorted):
        mesh = plsc.VectorSubcoreMesh(core_axis_name="c",
                                      subcore_axis_name="s")
        run = functools.partial(
            pl.kernel, mesh=mesh,
            out_type=jax.ShapeDtypeStruct((n, _C), _F32),
            scratch_types=[
                pltpu.VMEM((ch,), jnp.int32),
                pltpu.VMEM((ch, _C), _F32),
                pltpu.SemaphoreType.DMA,
            ])(body)
        return run(pos2d, out_sorted)

    return call


_NH = _N // 2
_K2H = _make_k2(_NH)
_K3H = _make_k3(_NH)
_K4H = _make_k4(_NH)
_K5H = _make_k5(_NH)


def kernel(x, v_first, capital_shares, step, warmup_steps, ln1_g, ln1_b, ln2_g, ln2_b, Wr, Wk, Wv, Wo, Ws, conf_W, W1, W2, Wmix, Wd, Wa, Wb1, Wb2):
    C = _C
    x2d = x.reshape(_N, C)
    vf2d = v_first.reshape(_N, C)
    Wd8 = jnp.pad(Wd, ((0, 0), (0, 7)))
    lnp = jnp.stack([ln1_g, ln1_b, ln2_g, ln2_b])
    cap = capital_shares[None, :]

    def k1_half(lo):
        return _k1(x2d[lo:lo + _NH], vf2d[lo:lo + _NH], Wr, Wk, Wv, Wo, Ws,
                   conf_W.T, Wa, Wd8, Wb1, Wb2, lnp, cap)

    x1A, hsA, ssA, winA, costA, sseA = k1_half(0)
    schedA = _K2H(winA[:, 0].reshape(_NH // _C2, _C2))
    x1B, hsB, ssB, winB, costB, sseB = k1_half(_NH)
    posA = schedA[0]
    srtA = _K3H(posA.reshape(_NW, 4, _NH // _NW // 4), hsA, ssA, x1A)
    schedB = _K2H(winB[:, 0].reshape(_NH // _C2, _C2))
    posB = schedB[0]
    srtB = _K3H(posB.reshape(_NW, 4, _NH // _NW // 4), hsB, ssB, x1B)
    outsA = _K4H(*schedA[2:], schedA[1], *srtA, W1, W2, Wmix)
    outA = _K5H(posA.reshape(_NW, _NH // _NW), outsA)
    outsB = _K4H(*schedB[2:], schedB[1], *srtB, W1, W2, Wmix)
    outB = _K5H(posB.reshape(_NW, _NH // _NW), outsB)

    out = jnp.concatenate([outA, outB], axis=0).reshape(_B, _T, C)
    winners = jnp.concatenate([winA[:, 0], winB[:, 0]]).reshape(_B, _T)
    costs = jnp.concatenate([costA[:, 0], costB[:, 0]]).reshape(_B, _T)
    recon_loss = (sseA[0, 0] + sseB[0, 0]) / (_N * C)
    return (out, v_first, winners, costs, recon_loss)


# speedup vs baseline: 1.2777x; 1.2777x over previous
import functools

import jax
import jax.numpy as jnp
from jax import lax
from jax.experimental import pallas as pl
from jax.experimental.pallas import tpu as pltpu
from jax.experimental.pallas import tpu_sc as plsc

_B, _T, _C = 2, 2048, 768
_NUM_RWKV, _NUM_TRANS = 6, 2
_E = _NUM_RWKV + _NUM_TRANS
_N = _B * _T

_BLK1 = 512
_NB1 = _N // _BLK1

_F32 = jnp.float32
_BF16 = jnp.bfloat16


def _bdot(a, b):
    return lax.dot_general(a, b, (((1,), (0,)), ((), ())),
                           preferred_element_type=_F32)


def _ddot(a, b):
    return lax.dot_general(a, b, (((1,), (0,)), ((), ())),
                           precision=lax.Precision.DEFAULT,
                           preferred_element_type=_F32)


def _ln_rows(x, g, b):
    mu = jnp.mean(x, axis=1, keepdims=True)
    d = x - mu
    var = jnp.mean(d * d, axis=1, keepdims=True)
    return d / jnp.sqrt(var + 1e-5) * g + b


def _k1_body(x_ref, vf_ref, wr_ref, wk_ref, wv_ref, wo_ref, ws_ref,
             cw_ref, wa_ref, wd_ref, wb1_ref, wb2_ref,
             lnp_ref, cap_ref,
             x1_ref, hs_ref, ss_ref, win_ref, cost_ref, sse_ref):
    i = pl.program_id(0)
    xf = x_ref[...]
    xn = _ln_rows(xf, lnp_ref[0:1, :], lnp_ref[1:2, :])
    r = _ddot(xn, wr_ref[...])
    k = _ddot(xn, wk_ref[...])
    v0 = _ddot(xn, wv_ref[...])
    v = v0 + (vf_ref[...] - v0) * jax.nn.sigmoid(k)
    att = _ddot(jax.nn.sigmoid(r) * v, wo_ref[...])
    state = jnp.tanh(_ddot(xn, ws_ref[...]))
    x1 = xf + att
    x1_ref[...] = x1
    h = _ln_rows(x1, lnp_ref[2:3, :], lnp_ref[3:4, :])
    conf = jax.nn.sigmoid(_ddot(h, cw_ref[...]))
    aff = _ddot(h, wa_ref[...])
    dcol = _ddot(h, wd_ref[...])[:, 0:1]
    diff = jax.nn.softplus(dcol)
    eff = conf * diff + 0.1 * aff + jnp.log(cap_ref[...] + 1e-6)
    costs = jnp.max(eff, axis=1)
    ids = lax.broadcasted_iota(jnp.int32, (_BLK1, _E), 1)
    win = jnp.min(jnp.where(eff == costs[:, None], ids, _E), axis=1)
    win_ref[...] = win[:, None]
    cost_ref[...] = costs[:, None]
    wc = jnp.sum(jnp.where(ids == win[:, None], conf, 0.0), axis=1)
    scale = (wc / (wc + 1e-6))[:, None]
    hs_ref[...] = h * scale
    ss_ref[...] = state * scale
    t1 = jnp.tanh(_ddot(state, wb1_ref[...]))
    recon = _ddot(t1, wb2_ref[...])
    dsse = jnp.sum((recon - h) ** 2)

    @pl.when(i == 0)
    def _():
        sse_ref[0, 0] = dsse

    @pl.when(i != 0)
    def _():
        sse_ref[0, 0] += dsse


def _k1(x2d, vf2d, Wr, Wk, Wv, Wo, Ws, cwT, Wa, Wd8, Wb1, Wb2, lnp, cap):
    row_spec = pl.BlockSpec((_BLK1, _C), lambda i: (i, 0))
    full = lambda a: pl.BlockSpec(a.shape, lambda i: tuple(0 for _ in a.shape))
    out_shapes = (
        jax.ShapeDtypeStruct((_N, _C), _F32),
        jax.ShapeDtypeStruct((_N, _C), _F32),
        jax.ShapeDtypeStruct((_N, _C), _F32),
        jax.ShapeDtypeStruct((_N, 1), jnp.int32),
        jax.ShapeDtypeStruct((_N, 1), _F32),
        jax.ShapeDtypeStruct((1, 1), _F32),
    )
    out_specs = (
        row_spec,
        row_spec,
        row_spec,
        pl.BlockSpec((_BLK1, 1), lambda i: (i, 0)),
        pl.BlockSpec((_BLK1, 1), lambda i: (i, 0)),
        pl.BlockSpec((1, 1), lambda i: (0, 0), memory_space=pltpu.SMEM),
    )
    ws = [Wr, Wk, Wv, Wo, Ws, cwT, Wa, Wd8, Wb1, Wb2, lnp, cap]
    return pl.pallas_call(
        _k1_body,
        grid=(_NB1,),
        in_specs=[row_spec, row_spec] + [full(w) for w in ws],
        out_specs=out_specs,
        out_shape=out_shapes,
        compiler_params=pltpu.CompilerParams(
            dimension_semantics=("arbitrary",)),
    )(x2d, vf2d, *ws)


_R2, _C2 = 32, 128
_M4 = 256
_NB4 = _N // _M4
_G4 = _NB4 + _E - 1
_G4P = _G4 + 1


def _k2_body(w_ref, pos_ref, offs_ref, bm_ref, exc_ref, wmx_ref,
             first_ref, exr_ref):
    w = w_ref[...]
    iu = lax.broadcasted_iota(jnp.int32, (_C2, _C2), 0)
    ju = lax.broadcasted_iota(jnp.int32, (_C2, _C2), 1)
    U = (iu < ju).astype(_BF16)
    il = lax.broadcasted_iota(jnp.int32, (_R2, _R2), 0)
    jl = lax.broadcasted_iota(jnp.int32, (_R2, _R2), 1)
    L = (jl < il).astype(_BF16)
    pos = jnp.zeros((_R2, _C2), _F32)
    off = jnp.float32(0.0)
    for e in range(_E):
        m = (w == e).astype(_F32)
        pref = _bdot(m.astype(_BF16), U)
        tot = jnp.sum(m, axis=1, keepdims=True)
        rowpref = _bdot(L, tot.astype(_BF16))
        offs_ref[e] = off.astype(jnp.int32)
        pos = pos + m * (off + rowpref + pref)
        off = off + jnp.sum(m)
    for e in range(_E, 16):
        offs_ref[e] = jnp.int32(_N)
    pos_ref[...] = pos.astype(jnp.int32)

    def body(t, carry):
        idx, prevb = carry
        b = t // _E
        e = t - b * _E
        lo = offs_ref[e]
        hi = offs_ref[e + 1]
        inc = jnp.logical_and(lo < (b + 1) * _M4, hi > b * _M4)

        @pl.when(inc)
        def _():
            bm_ref[idx] = b
            exr_ref[idx] = e
            exc_ref[idx] = e
            wmx_ref[idx] = jnp.clip(e - _NUM_RWKV, 0, 1)
            first_ref[idx] = jnp.where(b != prevb, 1, 0)

        return idx + inc.astype(jnp.int32), jnp.where(inc, b, prevb)

    idx, _ = lax.fori_loop(0, _NB4 * _E, body,
                           (jnp.int32(0), jnp.int32(-1)))

    def pad(g, c):
        bm_ref[g] = _NB4 - 1
        exr_ref[g] = _E
        exc_ref[g] = _E - 1
        wmx_ref[g] = 0
        first_ref[g] = 0
        return c

    lax.fori_loop(idx, _G4P, pad, jnp.int32(0))


def _k2(win2d32):
    smem1d = lambda n: pl.BlockSpec((n,), lambda i: (0,),
                                    memory_space=pltpu.SMEM)
    return pl.pallas_call(
        _k2_body,
        grid=(1,),
        in_specs=[pl.BlockSpec((_R2, _C2), lambda i: (0, 0))],
        out_specs=(
            pl.BlockSpec((_R2, _C2), lambda i: (0, 0)),
            smem1d(16), smem1d(_G4P), smem1d(_G4P), smem1d(_G4P),
            smem1d(_G4P), smem1d(_G4P),
        ),
        out_shape=(
            jax.ShapeDtypeStruct((_R2, _C2), jnp.int32),
            jax.ShapeDtypeStruct((16,), jnp.int32),
            jax.ShapeDtypeStruct((_G4P,), jnp.int32),
            jax.ShapeDtypeStruct((_G4P,), jnp.int32),
            jax.ShapeDtypeStruct((_G4P,), jnp.int32),
            jax.ShapeDtypeStruct((_G4P,), jnp.int32),
            jax.ShapeDtypeStruct((_G4P,), jnp.int32),
        ),
    )(win2d32)


def _k4_body(bm_r, exc_r, wmx_r, first_r, exraw_r, offs_r,
             hs_r, ss_r, x1_r, w1_r, w2_r, wm_r, out_r, acc_r):
    g = pl.program_id(0)
    e = exraw_r[g]
    lo = offs_r[e]
    hi = offs_r[e + 1]
    istrans = e >= _NUM_RWKV

    @pl.when(istrans)
    def _():
        acc_r[...] = hs_r[...] + _ddot(ss_r[...], wm_r[0])

    @pl.when(jnp.logical_not(istrans))
    def _():
        acc_r[...] = hs_r[...]

    u = jax.nn.relu(_ddot(acc_r[...], w1_r[0]))
    eo = _ddot(u, w2_r[0])
    rows = bm_r[g] * _M4 + lax.broadcasted_iota(jnp.int32, (_M4, 1), 0)
    inb = jnp.logical_and(rows >= lo, rows < hi)
    contrib = jnp.where(inb, eo, 0.0)
    isfirst = first_r[g] == 1

    @pl.when(isfirst)
    def _():
        out_r[...] = x1_r[...] + contrib

    @pl.when(jnp.logical_not(isfirst))
    def _():
        out_r[...] += contrib


def _k4(bm, exc, wmx, first, exraw, offs, hs_s, ss_s, x1_s, W1b, W2b, Wmb):
    row_spec = pl.BlockSpec((_M4, _C), lambda g, *s: (s[0][g], 0))
    grid_spec = pltpu.PrefetchScalarGridSpec(
        num_scalar_prefetch=6,
        grid=(_G4,),
        in_specs=[
            row_spec, row_spec, row_spec,
            pl.BlockSpec((1, _C, _C), lambda g, *s: (s[1][g], 0, 0)),
            pl.BlockSpec((1, _C, _C), lambda g, *s: (s[1][g], 0, 0)),
            pl.BlockSpec((1, _C, _C), lambda g, *s: (s[2][g], 0, 0)),
        ],
        out_specs=pl.BlockSpec((_M4, _C), lambda g, *s: (s[0][g], 0)),
        scratch_shapes=[pltpu.VMEM((_M4, _C), _F32)],
    )
    return pl.pallas_call(
        _k4_body,
        grid_spec=grid_spec,
        out_shape=jax.ShapeDtypeStruct((_N, _C), _F32),
        compiler_params=pltpu.CompilerParams(
            dimension_semantics=("arbitrary",)),
    )(bm, exc, wmx, first, exraw, offs, hs_s, ss_s, x1_s, W1b, W2b, Wmb)


_NC, _NS = 2, 16
_NW = _NC * _NS
_CH = _N // _NW
_QC = _CH // 4


def _k3_body(pos_r, hs_r, ss_r, x1_r, hs_o, ss_o, x1_o,
             idx_v, b0, b1, f0, f1, sib0, sib1, sob0, sob1,
             sif0, sif1, sof0, sof1):
    wid = lax.axis_index("s") * _NC + lax.axis_index("c")
    base = wid * _CH
    pltpu.sync_copy(pos_r.at[wid], idx_v)
    bbufs = (b0, b1)
    fbufs = (f0, f1)

    def inb(t):
        a = (hs_r, ss_r)[t // 4]
        return pltpu.make_async_copy(
            a.at[pl.ds(base + (t % 4) * _QC, _QC)],
            bbufs[t % 2], (sib0, sib1)[t % 2])

    def outb(t):
        d = (hs_o, ss_o)[t // 4]
        return pltpu.make_async_copy(
            bbufs[t % 2], d.at[idx_v.at[t % 4]], (sob0, sob1)[t % 2])

    def inf(t):
        return pltpu.make_async_copy(
            x1_r.at[pl.ds(base + t * _QC, _QC)],
            fbufs[t % 2], (sif0, sif1)[t % 2])

    def outf(t):
        return pltpu.make_async_copy(
            fbufs[t % 2], x1_o.at[idx_v.at[t]], (sof0, sof1)[t % 2])

    inb(0).start()
    inf(0).start()
    for t in range(8):
        inb(t).wait()
        if t >= 1:
            outb(t - 1).wait()
        if t + 1 < 8:
            inb(t + 1).start()
        outb(t).start()
    for t in range(4):
        inf(t).wait()
        if t >= 1:
            outf(t - 1).wait()
        if t + 1 < 4:
            inf(t + 1).start()
        outf(t).start()
    outb(7).wait()
    outf(3).wait()


def _k3(pos3d, hs, ss, x1):
    mesh = plsc.VectorSubcoreMesh(core_axis_name="c", subcore_axis_name="s")
    sdsf = jax.ShapeDtypeStruct((_N, _C), _F32)
    run = functools.partial(
        pl.kernel, mesh=mesh,
        out_type=(sdsf, sdsf, sdsf),
        scratch_types=[
            pltpu.VMEM((4, _QC), jnp.int32),
            pltpu.VMEM((_QC, _C), _F32),
            pltpu.VMEM((_QC, _C), _F32),
            pltpu.VMEM((_QC, _C), _F32),
            pltpu.VMEM((_QC, _C), _F32),
        ] + [pltpu.SemaphoreType.DMA] * 8)(_k3_body)
    return run(pos3d, hs, ss, x1)


def _k5_body(pos_r, os_r, xo_r, idx_v, buf, sem):
    wid = lax.axis_index("s") * _NC + lax.axis_index("c")
    base = wid * _CH
    pltpu.sync_copy(pos_r.at[wid], idx_v)
    pltpu.async_copy(os_r.at[idx_v], buf, sem).wait()
    pltpu.sync_copy(buf, xo_r.at[pl.ds(base, _CH)])


def _k5(pos2d, out_sorted):
    mesh = plsc.VectorSubcoreMesh(core_axis_name="c", subcore_axis_name="s")
    run = functools.partial(
        pl.kernel, mesh=mesh,
        out_type=jax.ShapeDtypeStruct((_N, _C), _F32),
        scratch_types=[
            pltpu.VMEM((_CH,), jnp.int32),
            pltpu.VMEM((_CH, _C), _F32),
            pltpu.SemaphoreType.DMA,
        ])(_k5_body)
    return run(pos2d, out_sorted)


def kernel(x, v_first, capital_shares, step, warmup_steps, ln1_g, ln1_b, ln2_g, ln2_b, Wr, Wk, Wv, Wo, Ws, conf_W, W1, W2, Wmix, Wd, Wa, Wb1, Wb2):
    C = _C
    x2d = x.reshape(_N, C)
    vf2d = v_first.reshape(_N, C)
    Wd8 = jnp.pad(Wd, ((0, 0), (0, 7)))
    lnp = jnp.stack([ln1_g, ln1_b, ln2_g, ln2_b])
    x1, hs, ss, win2d, cost2d, sse = _k1(
        x2d, vf2d, Wr, Wk, Wv, Wo, Ws,
        conf_W.T, Wa, Wd8, Wb1, Wb2, lnp, capital_shares[None, :])
    winners = win2d[:, 0]
    costs = cost2d[:, 0]
    recon_loss = sse[0, 0] / (_N * C)

    pos2d, offs, bm, exc, wmx, first, exraw = _k2(win2d.reshape(_R2, _C2))
    hs_s, ss_s, x1_s = _k3(pos2d.reshape(_NW, 4, _QC), hs, ss, x1)
    out_sorted = _k4(bm, exc, wmx, first, exraw, offs,
                     hs_s, ss_s, x1_s, W1, W2, Wmix)
    out = _k5(pos2d.reshape(_NW, _CH), out_sorted).reshape(_B, _T, C)
    return (out, v_first, winners.reshape(_B, _T), costs.reshape(_B, _T),
            recon_loss)
